# Initial kernel scaffold; baseline (speedup 1.0000x reference)
import jax
import jax.numpy as jnp
from jax.experimental import pallas as pl


def kernel(x, edge_index, edge_attr, pos, params):
    raise NotImplementedError("write your pallas kernel here")


# speedup vs baseline: 1.5171x; 1.5171x over previous
import functools

import jax
import jax.numpy as jnp
from jax.experimental import pallas as pl
from jax.experimental.pallas import tpu as pltpu

N = 2048
E = 65536
D = 256
H = 8
HD = D // H
EB = 1024
NB = E // EB

_F32 = jnp.float32


def _bn(x, w, b, eps=1e-5):
    mean = jnp.mean(x, axis=0, keepdims=True)
    xc = x - mean
    var = jnp.mean(xc * xc, axis=0, keepdims=True)
    return xc / jnp.sqrt(var + eps) * w + b


def _conv_kernel(row_ref, col_ref, x_ref, pos_ref,
                 n1w_ref, n1b_ref, n2w_ref, n2b_ref,
                 e1w_ref, e1b_ref, e2w_ref, e2b_ref,
                 oaw_ref, oxw_ref, o1b_ref, o2w_ref, o2b_ref,
                 bnw_ref, bnb_ref, nw_ref, nb_ref,
                 out_ref, xt_ref, agg_ref, cnt_ref):
    i = pl.program_id(0)

    @pl.when(i == 0)
    def _init():
        xh = jax.nn.silu(
            jnp.dot(x_ref[...], n1w_ref[...], preferred_element_type=_F32)
            + n1b_ref[...])
        xt_ref[...] = (jnp.dot(xh, n2w_ref[...], preferred_element_type=_F32)
                       + n2b_ref[...])
        agg_ref[...] = jnp.zeros_like(agg_ref)
        cnt_ref[...] = jnp.zeros_like(cnt_ref)

    row = row_ref[...]
    col = col_ref[...]
    iota = jax.lax.broadcasted_iota(jnp.int32, (EB, N), 1)
    rowoh = (row == iota).astype(_F32)
    coloh = (col == iota).astype(_F32)

    pos = pos_ref[...]
    rel = jnp.dot(rowoh - coloh, pos, preferred_element_type=_F32)
    rx, ry, rz = rel[:, 0:1], rel[:, 1:2], rel[:, 2:3]
    el = jnp.sqrt(rx * rx + ry * ry + rz * rz + 1e-12)
    zmask = el < 1e-10
    inv = 1.0 / jnp.where(zmask, jnp.ones_like(el), el)
    dx = jnp.where(zmask, 1.0, rx * inv)
    dy = jnp.where(zmask, 0.0, ry * inv)
    dz = jnp.where(zmask, 0.0, rz * inv)
    dn = 1.0 / (jnp.sqrt(dx * dx + dy * dy + dz * dz) + 1e-10)
    dx, dy, dz = dx * dn, dy * dn, dz * dn
    dx = jnp.nan_to_num(dx, nan=0.0, posinf=0.0, neginf=0.0)
    dy = jnp.nan_to_num(dy, nan=0.0, posinf=0.0, neginf=0.0)
    dz = jnp.nan_to_num(dz, nan=0.0, posinf=0.0, neginf=0.0)

    sph = (jnp.full_like(dx, 0.28209479177387814),
           0.4886025119029199 * dx, 0.4886025119029199 * dy,
           0.4886025119029199 * dz,
           1.0925484305920792 * dx * dy,
           1.0925484305920792 * dy * dz,
           0.31539156525252005 * (3.0 * dz * dz - 1.0),
           1.0925484305920792 * dx * dz,
           0.5462742152960396 * (dx * dx - dy * dy))
    e1w = e1w_ref[...]
    h1 = sph[0] * e1w[0:1, :] + e1b_ref[...]
    for k in range(1, 9):
        h1 = h1 + sph[k] * e1w[k:k + 1, :]
    h1 = jax.nn.silu(h1)
    e = jnp.dot(h1, e2w_ref[...], preferred_element_type=_F32) + e2b_ref[...]

    xtg = jnp.dot(rowoh, xt_ref[...], preferred_element_type=_F32)
    msgs = xtg * e

    agg_ref[...] += jax.lax.dot_general(
        coloh, msgs, (((0,), (0,)), ((), ())), preferred_element_type=_F32)
    cnt_ref[...] += jax.lax.dot_general(
        coloh, jnp.ones((EB, 1), _F32), (((0,), (0,)), ((), ())),
        preferred_element_type=_F32)

    @pl.when(i == NB - 1)
    def _finalize():
        cnt = jnp.maximum(cnt_ref[...], 1.0)
        agg = agg_ref[...] / cnt
        g1 = jax.nn.silu(
            jnp.dot(agg, oaw_ref[...], preferred_element_type=_F32)
            + jnp.dot(x_ref[...], oxw_ref[...], preferred_element_type=_F32)
            + o1b_ref[...])
        out = (jnp.dot(g1, o2w_ref[...], preferred_element_type=_F32)
               + o2b_ref[...])
        out = _bn(out, bnw_ref[...], bnb_ref[...])
        out_ref[...] = jax.nn.silu(_bn(out, nw_ref[...], nb_ref[...]))


def _attn_kernel(h_ref, wq_ref, bq_ref, wk_ref, bk_ref, wv_ref, bv_ref,
                 wo_ref, bo_ref, out_ref):
    i = pl.program_id(0)
    h = h_ref[...]
    q = jnp.dot(h, wq_ref[0], preferred_element_type=_F32) + bq_ref[0]
    k = jnp.dot(h, wk_ref[0], preferred_element_type=_F32) + bk_ref[0]
    v = jnp.dot(h, wv_ref[0], preferred_element_type=_F32) + bv_ref[0]
    s = jax.lax.dot_general(q, k, (((1,), (1,)), ((), ())),
                            preferred_element_type=_F32) * (HD ** -0.5)
    p = jax.nn.softmax(s, axis=-1)
    o = jnp.dot(p, v, preferred_element_type=_F32)
    contrib = jnp.dot(o, wo_ref[0], preferred_element_type=_F32)

    @pl.when(i == 0)
    def _first():
        out_ref[...] = contrib + bo_ref[...]

    @pl.when(i > 0)
    def _rest():
        out_ref[...] += contrib


def _full(shape):
    return pl.BlockSpec(shape, lambda i: (0,) * len(shape))


def _conv_call(row, col, x, pos_p, w):
    full2 = lambda a: _full(a.shape)
    in_specs = [
        pl.BlockSpec((EB, 1), lambda i: (i, 0)),
        pl.BlockSpec((EB, 1), lambda i: (i, 0)),
        _full((N, D)), _full((N, 128)),
    ] + [full2(a) for a in w]
    return pl.pallas_call(
        _conv_kernel,
        grid=(NB,),
        in_specs=in_specs,
        out_specs=_full((N, D)),
        out_shape=jax.ShapeDtypeStruct((N, D), _F32),
        scratch_shapes=[pltpu.VMEM((N, D), _F32),
                        pltpu.VMEM((N, D), _F32),
                        pltpu.VMEM((N, 1), _F32)],
    )(row, col, x, pos_p, *w)


def _attn_call(h, p):
    wq = p["attn_q_w"].reshape(D, H, HD).transpose(1, 0, 2)
    wk = p["attn_k_w"].reshape(D, H, HD).transpose(1, 0, 2)
    wv = p["attn_v_w"].reshape(D, H, HD).transpose(1, 0, 2)
    bq = p["attn_q_b"].reshape(H, 1, HD)
    bk = p["attn_k_b"].reshape(H, 1, HD)
    bv = p["attn_v_b"].reshape(H, 1, HD)
    wo = p["attn_o_w"].reshape(H, HD, D)
    bo = p["attn_o_b"].reshape(1, D)
    hw = pl.BlockSpec((1, D, HD), lambda i: (i, 0, 0))
    hb = pl.BlockSpec((1, 1, HD), lambda i: (i, 0, 0))
    ho = pl.BlockSpec((1, HD, D), lambda i: (i, 0, 0))
    return pl.pallas_call(
        _attn_kernel,
        grid=(H,),
        in_specs=[_full((N, D)), hw, hb, hw, hb, hw, hb, ho, _full((1, D))],
        out_specs=_full((N, D)),
        out_shape=jax.ShapeDtypeStruct((N, D), _F32),
    )(h, wq, bq, wk, bk, wv, bv, wo, bo)


def _conv_weights(p, prefix):
    b = lambda name: p[prefix + name + "_b"].reshape(1, D)
    o1w = p[prefix + "_out1_w"]
    return (p[prefix + "_node1_w"], b("_node1"),
            p[prefix + "_node2_w"], b("_node2"),
            p[prefix + "_edge1_w"], b("_edge1"),
            p[prefix + "_edge2_w"], b("_edge2"),
            o1w[:D], o1w[D:], b("_out1"),
            p[prefix + "_out2_w"], b("_out2"))


def kernel(x, edge_index, edge_attr, pos, params):
    p = params
    row = edge_index[0].reshape(E, 1)
    col = edge_index[1].reshape(E, 1)
    pos_p = jnp.pad(pos, ((0, 0), (0, 128 - pos.shape[1])))

    bn1 = (p["conv1_bn_w"].reshape(1, D), p["conv1_bn_b"].reshape(1, D),
           p["norm1_w"].reshape(1, D), p["norm1_b"].reshape(1, D))
    bn2 = (p["conv2_bn_w"].reshape(1, D), p["conv2_bn_b"].reshape(1, D),
           p["norm2_w"].reshape(1, D), p["norm2_b"].reshape(1, D))

    h = _conv_call(row, col, x, pos_p, _conv_weights(p, "conv1") + bn1)
    h = _conv_call(row, col, h, pos_p, _conv_weights(p, "conv2") + bn2)
    return _attn_call(h, p)
